# Initial kernel scaffold; baseline (speedup 1.0000x reference)
import jax
import jax.numpy as jnp
from jax.experimental import pallas as pl


def kernel(raw_logits, target_left, target_right, pointer_labels):
    raise NotImplementedError("write your pallas kernel here")


# speedup vs baseline: 5.5206x; 5.5206x over previous
import functools
import math

import jax
import jax.numpy as jnp
from jax import lax
from jax.experimental import pallas as pl
from jax.experimental.pallas import tpu as pltpu
from jax.experimental.pallas import tpu_sc as plsc

L = 16
NC, NS = 2, 16
NW = NC * NS
G = 4

LOG_EPS = math.log(1e-7)


def _sc_gather_body(S, rows_per_w, raw_hbm, tr_hbm, a_hbm, b_hbm,
                    row_v, idx_v, a_v, b_v):
    wid = lax.axis_index("s") * NC + lax.axis_index("c")
    n_chunks = S // L

    S2 = 2 * S

    def group_body(gi, carry):
        r0 = wid * rows_per_w + gi * G
        for g in range(G):
            pltpu.sync_copy(raw_hbm.at[r0 + g], row_v.at[pl.ds(g * S2, S2)])
            pltpu.sync_copy(tr_hbm.at[r0 + g], idx_v.at[pl.ds(g * S, S)])
        for g in range(G):

            def chunk_body(c, _):
                idx = idx_v[pl.ds(g * S + c * L, L)]
                a_v[pl.ds(g * S + c * L, L)] = plsc.load_gather(
                    row_v, [idx * 2 + (g * S2)])
                iota = lax.iota(jnp.int32, L)
                b_v[pl.ds(g * S + c * L, L)] = plsc.load_gather(
                    row_v, [iota * 2 + (g * S2 + 2 * L * c + 1)])
                return 0

            lax.fori_loop(0, n_chunks, chunk_body, 0, unroll=4)
        for g in range(G):
            pltpu.sync_copy(a_v.at[pl.ds(g * S, S)], a_hbm.at[r0 + g])
            pltpu.sync_copy(b_v.at[pl.ds(g * S, S)], b_hbm.at[r0 + g])
        return carry

    lax.fori_loop(0, rows_per_w // G, group_body, 0)


def _sc_gather(raw2, tr2):
    R, S2 = raw2.shape
    S = S2 // 2
    rows_per_w = R // NW
    mesh = plsc.VectorSubcoreMesh(core_axis_name="c", subcore_axis_name="s")
    f = pl.kernel(
        functools.partial(_sc_gather_body, S, rows_per_w),
        out_type=[
            jax.ShapeDtypeStruct((R, S), jnp.float32),
            jax.ShapeDtypeStruct((R, S), jnp.float32),
        ],
        mesh=mesh,
        compiler_params=pltpu.CompilerParams(needs_layout_passes=False),
        scratch_types=[
            pltpu.VMEM((G * S2,), jnp.float32),
            pltpu.VMEM((G * S,), jnp.int32),
            pltpu.VMEM((G * S,), jnp.float32),
            pltpu.VMEM((G * S,), jnp.float32),
        ],
    )
    return f(raw2, tr2)


def _tc_loss_body(a_ref, b_ref, lab_ref, out_ref):
    a = a_ref[...]
    b = b_ref[...]
    m = jnp.max(jnp.maximum(a, b), axis=1, keepdims=True)
    q = jnp.exp(a - m) + jnp.exp(b - m)
    t = jnp.sum(q, axis=1, keepdims=True)
    logp = jnp.log(q) - jnp.log(t)
    logp = jnp.clip(logp, LOG_EPS, 0.0)
    out_ref[...] = -jnp.sum(lab_ref[...] * logp, axis=1, keepdims=True)


def _tc_loss(a, b, labels2):
    R, S = a.shape
    BR = 128
    grid = (R // BR,)
    return pl.pallas_call(
        _tc_loss_body,
        grid=grid,
        in_specs=[
            pl.BlockSpec((BR, S), lambda i: (i, 0)),
            pl.BlockSpec((BR, S), lambda i: (i, 0)),
            pl.BlockSpec((BR, S), lambda i: (i, 0)),
        ],
        out_specs=pl.BlockSpec((BR, 1), lambda i: (i, 0)),
        out_shape=jax.ShapeDtypeStruct((R, 1), jnp.float32),
    )(a, b, labels2)


def kernel(raw_logits, target_left, target_right, pointer_labels):
    del target_left
    B, S, S2 = raw_logits.shape
    R = B * S
    raw2 = raw_logits.reshape(R, S2)
    tr2 = target_right.reshape(R, S)
    lab2 = pointer_labels.reshape(R, S)
    a, b = _sc_gather(raw2, tr2)
    loss = _tc_loss(a, b, lab2)
    return loss.reshape(B, S)


# speedup vs baseline: 6.4191x; 1.1628x over previous
import functools
import math

import jax
import jax.numpy as jnp
from jax import lax
from jax.experimental import pallas as pl
from jax.experimental.pallas import tpu as pltpu
from jax.experimental.pallas import tpu_sc as plsc

L = 16
NC, NS = 2, 16
NW = NC * NS
G = 4

LOG_EPS = math.log(1e-7)


def _sc_gather_body(S, rows_per_w, raw_hbm, tr_hbm, a_hbm, b_hbm,
                    row_v0, idx_v0, a_v0, b_v0,
                    row_v1, idx_v1, a_v1, b_v1,
                    sin0, sout0, sin1, sout1):
    S2 = 2 * S
    wid = lax.axis_index("s") * NC + lax.axis_index("c")
    n_chunks = S // L
    n_groups = rows_per_w // G
    base_row = wid * rows_per_w

    def in_descs(g, row_v, idx_v, sem):
        r0 = base_row + g * G
        return (
            pltpu.make_async_copy(
                raw_hbm.at[pl.ds(r0 * S2, G * S2)], row_v, sem),
            pltpu.make_async_copy(
                tr_hbm.at[pl.ds(r0 * S, G * S)], idx_v, sem),
        )

    def out_descs(g, a_v, b_v, sem):
        r0 = base_row + g * G
        return (
            pltpu.make_async_copy(
                a_v, a_hbm.at[pl.ds(r0 * S, G * S)], sem),
            pltpu.make_async_copy(
                b_v, b_hbm.at[pl.ds(r0 * S, G * S)], sem),
        )

    def start(descs):
        for d in descs:
            d.start()

    def wait(descs):
        for d in descs:
            d.wait()

    def gather_group(row_v, idx_v, a_v, b_v):
        iota2 = lax.iota(jnp.int32, L) * 2
        for g in range(G):

            def chunk_body(c, _):
                o = g * S + c * L
                idx = idx_v[pl.ds(o, L)]
                a_v[pl.ds(o, L)] = plsc.load_gather(
                    row_v, [idx * 2 + (g * S2)])
                b_v[pl.ds(o, L)] = plsc.load_gather(
                    row_v, [iota2 + (g * S2 + 2 * L * c + 1)])
                return 0

            lax.fori_loop(0, n_chunks, chunk_body, 0, unroll=8)

    bufs = ((row_v0, idx_v0, a_v0, b_v0, sin0, sout0),
            (row_v1, idx_v1, a_v1, b_v1, sin1, sout1))

    start(in_descs(0, row_v0, idx_v0, sin0))

    def pair_body(p, carry):
        for k in (0, 1):
            g = 2 * p + k
            row_v, idx_v, a_v, b_v, sin, sout = bufs[k]
            nrow_v, nidx_v, _, _, nsin, _ = bufs[1 - k]

            @pl.when(g + 1 < n_groups)
            def _():
                start(in_descs(g + 1, nrow_v, nidx_v, nsin))

            wait(in_descs(g, row_v, idx_v, sin))

            @pl.when(g >= 2)
            def _():
                wait(out_descs(g - 2, a_v, b_v, sout))

            gather_group(row_v, idx_v, a_v, b_v)
            start(out_descs(g, a_v, b_v, sout))
        return carry

    lax.fori_loop(0, n_groups // 2, pair_body, 0)
    wait(out_descs(n_groups - 2, a_v0, b_v0, sout0))
    wait(out_descs(n_groups - 1, a_v1, b_v1, sout1))


def _sc_gather(raw2, tr2):
    R, S2 = raw2.shape
    S = S2 // 2
    rows_per_w = R // NW
    mesh = plsc.VectorSubcoreMesh(core_axis_name="c", subcore_axis_name="s")
    f = pl.kernel(
        functools.partial(_sc_gather_body, S, rows_per_w),
        out_type=[
            jax.ShapeDtypeStruct((R * S,), jnp.float32),
            jax.ShapeDtypeStruct((R * S,), jnp.float32),
        ],
        mesh=mesh,
        compiler_params=pltpu.CompilerParams(needs_layout_passes=False),
        scratch_types=[
            pltpu.VMEM((G * S2,), jnp.float32),
            pltpu.VMEM((G * S,), jnp.int32),
            pltpu.VMEM((G * S,), jnp.float32),
            pltpu.VMEM((G * S,), jnp.float32),
            pltpu.VMEM((G * S2,), jnp.float32),
            pltpu.VMEM((G * S,), jnp.int32),
            pltpu.VMEM((G * S,), jnp.float32),
            pltpu.VMEM((G * S,), jnp.float32),
            pltpu.SemaphoreType.DMA,
            pltpu.SemaphoreType.DMA,
            pltpu.SemaphoreType.DMA,
            pltpu.SemaphoreType.DMA,
        ],
    )
    a, b = f(raw2.reshape(-1), tr2.reshape(-1))
    return a.reshape(R, S), b.reshape(R, S)


def _tc_loss_body(a_ref, b_ref, lab_ref, out_ref):
    a = a_ref[...]
    b = b_ref[...]
    m = jnp.max(jnp.maximum(a, b), axis=1, keepdims=True)
    q = jnp.exp(a - m) + jnp.exp(b - m)
    t = jnp.sum(q, axis=1, keepdims=True)
    logp = jnp.log(q) - jnp.log(t)
    logp = jnp.clip(logp, LOG_EPS, 0.0)
    out_ref[...] = -jnp.sum(lab_ref[...] * logp, axis=1, keepdims=True)


def _tc_loss(a, b, labels2):
    R, S = a.shape
    BR = 128
    grid = (R // BR,)
    return pl.pallas_call(
        _tc_loss_body,
        grid=grid,
        in_specs=[
            pl.BlockSpec((BR, S), lambda i: (i, 0)),
            pl.BlockSpec((BR, S), lambda i: (i, 0)),
            pl.BlockSpec((BR, S), lambda i: (i, 0)),
        ],
        out_specs=pl.BlockSpec((BR, 1), lambda i: (i, 0)),
        out_shape=jax.ShapeDtypeStruct((R, 1), jnp.float32),
    )(a, b, labels2)


def kernel(raw_logits, target_left, target_right, pointer_labels):
    del target_left
    B, S, S2 = raw_logits.shape
    R = B * S
    raw2 = raw_logits.reshape(R, S2)
    tr2 = target_right.reshape(R, S)
    lab2 = pointer_labels.reshape(R, S)
    a, b = _sc_gather(raw2, tr2)
    loss = _tc_loss(a, b, lab2)
    return loss.reshape(B, S)


# speedup vs baseline: 10.9443x; 1.7049x over previous
import functools
import math

import jax
import jax.numpy as jnp
from jax import lax
from jax.experimental import pallas as pl
from jax.experimental.pallas import tpu as pltpu
from jax.experimental.pallas import tpu_sc as plsc

L = 16
NC, NS = 2, 16
NW = NC * NS
G = 4

LOG_EPS = math.log(1e-7)


def _sc_gather_body(S, rows_per_w, raw_hbm, tr_hbm, a_hbm, b_hbm,
                    row_v0, idx_v0, a_v0, b_v0,
                    row_v1, idx_v1, a_v1, b_v1,
                    sin0, sout0, sin1, sout1):
    S2 = 2 * S
    wid = lax.axis_index("s") * NC + lax.axis_index("c")
    n_chunks = S // L
    n_groups = rows_per_w // G
    base_row = wid * rows_per_w

    def in_descs(g, row_v, idx_v, sem):
        r0 = base_row + g * G
        return (
            pltpu.make_async_copy(
                raw_hbm.at[pl.ds(r0 * S2, G * S2)], row_v, sem),
            pltpu.make_async_copy(
                tr_hbm.at[pl.ds(r0 * S, G * S)], idx_v, sem),
        )

    def out_descs(g, a_v, b_v, sem):
        r0 = base_row + g * G
        return (
            pltpu.make_async_copy(
                a_v, a_hbm.at[pl.ds(r0 * S, G * S)], sem),
            pltpu.make_async_copy(
                b_v, b_hbm.at[pl.ds(r0 * S, G * S)], sem),
        )

    def start(descs):
        for d in descs:
            d.start()

    def wait(descs):
        for d in descs:
            d.wait()

    def gather_group(row_v, idx_v, a_v, b_v):
        iota2 = lax.iota(jnp.int32, L) * 2
        for g in range(G):

            @plsc.parallel_loop(0, n_chunks, unroll=8)
            def _(c):
                o = g * S + c * L
                idx = idx_v[pl.ds(o, L)]
                a_v[pl.ds(o, L)] = plsc.load_gather(
                    row_v, [idx * 2 + (g * S2)])
                b_v[pl.ds(o, L)] = plsc.load_gather(
                    row_v, [iota2 + (g * S2 + 2 * L * c + 1)])

    bufs = ((row_v0, idx_v0, a_v0, b_v0, sin0, sout0),
            (row_v1, idx_v1, a_v1, b_v1, sin1, sout1))

    start(in_descs(0, row_v0, idx_v0, sin0))

    def pair_body(p, carry):
        for k in (0, 1):
            g = 2 * p + k
            row_v, idx_v, a_v, b_v, sin, sout = bufs[k]
            nrow_v, nidx_v, _, _, nsin, _ = bufs[1 - k]

            @pl.when(g + 1 < n_groups)
            def _():
                start(in_descs(g + 1, nrow_v, nidx_v, nsin))

            wait(in_descs(g, row_v, idx_v, sin))

            @pl.when(g >= 2)
            def _():
                wait(out_descs(g - 2, a_v, b_v, sout))

            gather_group(row_v, idx_v, a_v, b_v)
            start(out_descs(g, a_v, b_v, sout))
        return carry

    lax.fori_loop(0, n_groups // 2, pair_body, 0)
    wait(out_descs(n_groups - 2, a_v0, b_v0, sout0))
    wait(out_descs(n_groups - 1, a_v1, b_v1, sout1))


def _sc_gather(raw2, tr2):
    R, S2 = raw2.shape
    S = S2 // 2
    rows_per_w = R // NW
    mesh = plsc.VectorSubcoreMesh(core_axis_name="c", subcore_axis_name="s")
    f = pl.kernel(
        functools.partial(_sc_gather_body, S, rows_per_w),
        out_type=[
            jax.ShapeDtypeStruct((R * S,), jnp.float32),
            jax.ShapeDtypeStruct((R * S,), jnp.float32),
        ],
        mesh=mesh,
        compiler_params=pltpu.CompilerParams(needs_layout_passes=False),
        scratch_types=[
            pltpu.VMEM((G * S2,), jnp.float32),
            pltpu.VMEM((G * S,), jnp.int32),
            pltpu.VMEM((G * S,), jnp.float32),
            pltpu.VMEM((G * S,), jnp.float32),
            pltpu.VMEM((G * S2,), jnp.float32),
            pltpu.VMEM((G * S,), jnp.int32),
            pltpu.VMEM((G * S,), jnp.float32),
            pltpu.VMEM((G * S,), jnp.float32),
            pltpu.SemaphoreType.DMA,
            pltpu.SemaphoreType.DMA,
            pltpu.SemaphoreType.DMA,
            pltpu.SemaphoreType.DMA,
        ],
    )
    a, b = f(raw2.reshape(-1), tr2.reshape(-1))
    return a.reshape(R, S), b.reshape(R, S)


def _tc_loss_body(a_ref, b_ref, lab_ref, out_ref):
    a = a_ref[...]
    b = b_ref[...]
    m = jnp.max(jnp.maximum(a, b), axis=1, keepdims=True)
    q = jnp.exp(a - m) + jnp.exp(b - m)
    t = jnp.sum(q, axis=1, keepdims=True)
    logp = jnp.log(q) - jnp.log(t)
    logp = jnp.clip(logp, LOG_EPS, 0.0)
    out_ref[...] = -jnp.sum(lab_ref[...] * logp, axis=1, keepdims=True)


def _tc_loss(a, b, labels2):
    R, S = a.shape
    BR = 128
    grid = (R // BR,)
    return pl.pallas_call(
        _tc_loss_body,
        grid=grid,
        in_specs=[
            pl.BlockSpec((BR, S), lambda i: (i, 0)),
            pl.BlockSpec((BR, S), lambda i: (i, 0)),
            pl.BlockSpec((BR, S), lambda i: (i, 0)),
        ],
        out_specs=pl.BlockSpec((BR, 1), lambda i: (i, 0)),
        out_shape=jax.ShapeDtypeStruct((R, 1), jnp.float32),
    )(a, b, labels2)


def kernel(raw_logits, target_left, target_right, pointer_labels):
    del target_left
    B, S, S2 = raw_logits.shape
    R = B * S
    raw2 = raw_logits.reshape(R, S2)
    tr2 = target_right.reshape(R, S)
    lab2 = pointer_labels.reshape(R, S)
    a, b = _sc_gather(raw2, tr2)
    loss = _tc_loss(a, b, lab2)
    return loss.reshape(B, S)


# speedup vs baseline: 24.0122x; 2.1940x over previous
import functools
import math

import jax
import jax.numpy as jnp
from jax import lax
from jax.experimental import pallas as pl
from jax.experimental.pallas import tpu as pltpu
from jax.experimental.pallas import tpu_sc as plsc

L = 16
NC, NS = 2, 16
NW = NC * NS
G = 4

LOG_EPS = math.log(1e-7)


def _sc_gather_body(S, rows_per_w, raw_hbm, tr_hbm, a_hbm, b_hbm,
                    row_v0, idx_v0, a_v0, b_v0,
                    row_v1, idx_v1, a_v1, b_v1,
                    sin0, sout0, sin1, sout1):
    S2 = 2 * S
    wid = lax.axis_index("s") * NC + lax.axis_index("c")
    n_chunks = S // L
    n_groups = rows_per_w // G
    base_row = wid * rows_per_w

    def in_descs(g, row_v, idx_v, sem):
        r0 = base_row + g * G
        return (
            pltpu.make_async_copy(raw_hbm.at[pl.ds(r0, G)], row_v, sem),
            pltpu.make_async_copy(tr_hbm.at[pl.ds(r0, G)], idx_v, sem),
        )

    def out_descs(g, a_v, b_v, sem):
        r0 = base_row + g * G
        return (
            pltpu.make_async_copy(a_v, a_hbm.at[pl.ds(r0, G)], sem),
            pltpu.make_async_copy(b_v, b_hbm.at[pl.ds(r0, G)], sem),
        )

    def start(descs):
        for d in descs:
            d.start()

    def wait(descs):
        for d in descs:
            d.wait()

    def gather_group(row_v, idx_v, a_v, b_v):
        iota2 = lax.iota(jnp.int32, L) * 2
        for g in range(G):
            gs = jnp.full((L,), g, dtype=jnp.int32)

            @plsc.parallel_loop(0, n_chunks, unroll=8)
            def _(c):
                o = c * L
                idx = idx_v[g, pl.ds(o, L)]
                a_v[g, pl.ds(o, L)] = plsc.load_gather(
                    row_v, [gs, idx * 2])
                b_v[g, pl.ds(o, L)] = plsc.load_gather(
                    row_v, [gs, iota2 + (2 * L * c + 1)])

    bufs = ((row_v0, idx_v0, a_v0, b_v0, sin0, sout0),
            (row_v1, idx_v1, a_v1, b_v1, sin1, sout1))

    start(in_descs(0, row_v0, idx_v0, sin0))

    def pair_body(p, carry):
        for k in (0, 1):
            g = 2 * p + k
            row_v, idx_v, a_v, b_v, sin, sout = bufs[k]
            nrow_v, nidx_v, _, _, nsin, _ = bufs[1 - k]

            @pl.when(g + 1 < n_groups)
            def _():
                start(in_descs(g + 1, nrow_v, nidx_v, nsin))

            wait(in_descs(g, row_v, idx_v, sin))

            @pl.when(g >= 2)
            def _():
                wait(out_descs(g - 2, a_v, b_v, sout))

            gather_group(row_v, idx_v, a_v, b_v)
            start(out_descs(g, a_v, b_v, sout))
        return carry

    lax.fori_loop(0, n_groups // 2, pair_body, 0)
    wait(out_descs(n_groups - 2, a_v0, b_v0, sout0))
    wait(out_descs(n_groups - 1, a_v1, b_v1, sout1))


def _sc_gather(raw2, tr2):
    R, S2 = raw2.shape
    S = S2 // 2
    rows_per_w = R // NW
    mesh = plsc.VectorSubcoreMesh(core_axis_name="c", subcore_axis_name="s")
    f = pl.kernel(
        functools.partial(_sc_gather_body, S, rows_per_w),
        out_type=[
            jax.ShapeDtypeStruct((R, S), jnp.float32),
            jax.ShapeDtypeStruct((R, S), jnp.float32),
        ],
        mesh=mesh,
        compiler_params=pltpu.CompilerParams(needs_layout_passes=False),
        scratch_types=[
            pltpu.VMEM((G, S2), jnp.float32),
            pltpu.VMEM((G, S), jnp.int32),
            pltpu.VMEM((G, S), jnp.float32),
            pltpu.VMEM((G, S), jnp.float32),
            pltpu.VMEM((G, S2), jnp.float32),
            pltpu.VMEM((G, S), jnp.int32),
            pltpu.VMEM((G, S), jnp.float32),
            pltpu.VMEM((G, S), jnp.float32),
            pltpu.SemaphoreType.DMA,
            pltpu.SemaphoreType.DMA,
            pltpu.SemaphoreType.DMA,
            pltpu.SemaphoreType.DMA,
        ],
    )
    return f(raw2, tr2)


def _tc_loss_body(a_ref, b_ref, lab_ref, out_ref):
    a = a_ref[...]
    b = b_ref[...]
    m = jnp.max(jnp.maximum(a, b), axis=1, keepdims=True)
    q = jnp.exp(a - m) + jnp.exp(b - m)
    t = jnp.sum(q, axis=1, keepdims=True)
    logp = jnp.log(q) - jnp.log(t)
    logp = jnp.clip(logp, LOG_EPS, 0.0)
    out_ref[...] = -jnp.sum(lab_ref[...] * logp, axis=1, keepdims=True)


def _tc_loss(a, b, labels2):
    R, S = a.shape
    BR = 128
    grid = (R // BR,)
    return pl.pallas_call(
        _tc_loss_body,
        grid=grid,
        in_specs=[
            pl.BlockSpec((BR, S), lambda i: (i, 0)),
            pl.BlockSpec((BR, S), lambda i: (i, 0)),
            pl.BlockSpec((BR, S), lambda i: (i, 0)),
        ],
        out_specs=pl.BlockSpec((BR, 1), lambda i: (i, 0)),
        out_shape=jax.ShapeDtypeStruct((R, 1), jnp.float32),
    )(a, b, labels2)


def kernel(raw_logits, target_left, target_right, pointer_labels):
    del target_left
    B, S, S2 = raw_logits.shape
    R = B * S
    raw2 = raw_logits.reshape(R, S2)
    tr2 = target_right.reshape(R, S)
    lab2 = pointer_labels.reshape(R, S)
    a, b = _sc_gather(raw2, tr2)
    loss = _tc_loss(a, b, lab2)
    return loss.reshape(B, S)


# speedup vs baseline: 24.9762x; 1.0401x over previous
import functools
import math

import jax
import jax.numpy as jnp
from jax import lax
from jax.experimental import pallas as pl
from jax.experimental.pallas import tpu as pltpu
from jax.experimental.pallas import tpu_sc as plsc

L = 16
NC, NS = 2, 16
NW = NC * NS
G = 4

LOG_EPS = math.log(1e-7)


def _sc_gather_body(S, rows_per_w, row0, raw_hbm, tr_hbm, a_hbm, b_hbm,
                    row_v0, idx_v0, a_v0, b_v0,
                    row_v1, idx_v1, a_v1, b_v1,
                    sin0, sout0, sin1, sout1):
    S2 = 2 * S
    wid = lax.axis_index("s") * NC + lax.axis_index("c")
    n_chunks = S // L
    n_groups = rows_per_w // G
    base_row = wid * rows_per_w

    def in_descs(g, row_v, idx_v, sem):
        r0 = row0 + base_row + g * G
        return (
            pltpu.make_async_copy(raw_hbm.at[pl.ds(r0, G)], row_v, sem),
            pltpu.make_async_copy(tr_hbm.at[pl.ds(r0, G)], idx_v, sem),
        )

    def out_descs(g, a_v, b_v, sem):
        r0 = base_row + g * G
        return (
            pltpu.make_async_copy(a_v, a_hbm.at[pl.ds(r0, G)], sem),
            pltpu.make_async_copy(b_v, b_hbm.at[pl.ds(r0, G)], sem),
        )

    def start(descs):
        for d in descs:
            d.start()

    def wait(descs):
        for d in descs:
            d.wait()

    def gather_group(row_v, idx_v, a_v, b_v):
        iota2 = lax.iota(jnp.int32, L) * 2
        for g in range(G):
            gs = jnp.full((L,), g, dtype=jnp.int32)

            @plsc.parallel_loop(0, n_chunks, unroll=8)
            def _(c):
                o = c * L
                idx = idx_v[g, pl.ds(o, L)]
                a_v[g, pl.ds(o, L)] = plsc.load_gather(
                    row_v, [gs, idx * 2])
                b_v[g, pl.ds(o, L)] = plsc.load_gather(
                    row_v, [gs, iota2 + (2 * L * c + 1)])

    bufs = ((row_v0, idx_v0, a_v0, b_v0, sin0, sout0),
            (row_v1, idx_v1, a_v1, b_v1, sin1, sout1))

    start(in_descs(0, row_v0, idx_v0, sin0))

    def pair_body(p, carry):
        for k in (0, 1):
            g = 2 * p + k
            row_v, idx_v, a_v, b_v, sin, sout = bufs[k]
            nrow_v, nidx_v, _, _, nsin, _ = bufs[1 - k]

            @pl.when(g + 1 < n_groups)
            def _():
                start(in_descs(g + 1, nrow_v, nidx_v, nsin))

            wait(in_descs(g, row_v, idx_v, sin))

            @pl.when(g >= 2)
            def _():
                wait(out_descs(g - 2, a_v, b_v, sout))

            gather_group(row_v, idx_v, a_v, b_v)
            start(out_descs(g, a_v, b_v, sout))
        return carry

    lax.fori_loop(0, n_groups // 2, pair_body, 0)
    wait(out_descs(n_groups - 2, a_v0, b_v0, sout0))
    wait(out_descs(n_groups - 1, a_v1, b_v1, sout1))


def _sc_gather(raw2, tr2, row0, nrows):
    _, S2 = raw2.shape
    S = S2 // 2
    rows_per_w = nrows // NW
    mesh = plsc.VectorSubcoreMesh(core_axis_name="c", subcore_axis_name="s")
    f = pl.kernel(
        functools.partial(_sc_gather_body, S, rows_per_w, row0),
        out_type=[
            jax.ShapeDtypeStruct((nrows, S), jnp.float32),
            jax.ShapeDtypeStruct((nrows, S), jnp.float32),
        ],
        mesh=mesh,
        compiler_params=pltpu.CompilerParams(needs_layout_passes=False),
        scratch_types=[
            pltpu.VMEM((G, S2), jnp.float32),
            pltpu.VMEM((G, S), jnp.int32),
            pltpu.VMEM((G, S), jnp.float32),
            pltpu.VMEM((G, S), jnp.float32),
            pltpu.VMEM((G, S2), jnp.float32),
            pltpu.VMEM((G, S), jnp.int32),
            pltpu.VMEM((G, S), jnp.float32),
            pltpu.VMEM((G, S), jnp.float32),
            pltpu.SemaphoreType.DMA,
            pltpu.SemaphoreType.DMA,
            pltpu.SemaphoreType.DMA,
            pltpu.SemaphoreType.DMA,
        ],
    )
    return f(raw2, tr2)


def _tc_loss_body(a_ref, b_ref, lab_ref, out_ref):
    a = a_ref[...].astype(jnp.float32)
    b = b_ref[...].astype(jnp.float32)
    m = jnp.max(jnp.maximum(a, b), axis=1, keepdims=True)
    q = jnp.exp(a - m) + jnp.exp(b - m)
    t = jnp.sum(q, axis=1, keepdims=True)
    logp = jnp.log(q) - jnp.log(t)
    logp = jnp.clip(logp, LOG_EPS, 0.0)
    out_ref[...] = -jnp.sum(lab_ref[...] * logp, axis=1, keepdims=True)


def _tc_loss(a, b, labels2, row0):
    nrows, S = a.shape
    BR = 128
    grid = (nrows // BR,)
    off = row0 // BR
    return pl.pallas_call(
        _tc_loss_body,
        grid=grid,
        in_specs=[
            pl.BlockSpec((BR, S), lambda i: (i, 0)),
            pl.BlockSpec((BR, S), lambda i: (i, 0)),
            pl.BlockSpec((BR, S), lambda i: (i + off, 0)),
        ],
        out_specs=pl.BlockSpec((BR, 1), lambda i: (i, 0)),
        out_shape=jax.ShapeDtypeStruct((nrows, 1), jnp.float32),
    )(a, b, labels2)


NCHUNK = 4


def kernel(raw_logits, target_left, target_right, pointer_labels):
    del target_left
    B, S, S2 = raw_logits.shape
    R = B * S
    raw2 = raw_logits.reshape(R, S2)
    tr2 = target_right.reshape(R, S)
    lab2 = pointer_labels.reshape(R, S)
    cr = R // NCHUNK
    losses = []
    for ci in range(NCHUNK):
        a, b = _sc_gather(raw2, tr2, ci * cr, cr)
        losses.append(_tc_loss(a, b, lab2, ci * cr))
    loss = jnp.concatenate(losses, axis=0)
    return loss.reshape(B, S)


# speedup vs baseline: 30.0840x; 1.2045x over previous
import functools
import math

import jax
import jax.numpy as jnp
from jax import lax
from jax.experimental import pallas as pl
from jax.experimental.pallas import tpu as pltpu
from jax.experimental.pallas import tpu_sc as plsc

L = 16
NC, NS = 2, 16
NW = NC * NS
G = 2
NCHUNK = 4

LOG_EPS = math.log(1e-7)


def _sc_gather_body(S, half_rows, row0, hpw, raw_hbm, tr_hbm, wa_hbm, wb_hbm,
                    rlo_v0, rhi_v0, ilo_v0, ihi_v0, wa_v0, wb_v0,
                    rlo_v1, rhi_v1, ilo_v1, ihi_v1, wa_v1, wb_v1,
                    sin0, sout0, sin1, sout1):
    S2 = 2 * S
    wid = lax.axis_index("s") * NC + lax.axis_index("c")
    n_chunks = S // L
    n_groups = hpw // G
    base_word = wid * hpw

    def in_descs(g, rlo_v, rhi_v, ilo_v, ihi_v, sem):
        rlo = row0 + base_word + g * G
        rhi = rlo + half_rows
        return (
            pltpu.make_async_copy(raw_hbm.at[pl.ds(rlo, G)], rlo_v, sem),
            pltpu.make_async_copy(raw_hbm.at[pl.ds(rhi, G)], rhi_v, sem),
            pltpu.make_async_copy(tr_hbm.at[pl.ds(rlo, G)], ilo_v, sem),
            pltpu.make_async_copy(tr_hbm.at[pl.ds(rhi, G)], ihi_v, sem),
        )

    def out_descs(g, wa_v, wb_v, sem):
        u0 = base_word + g * G
        return (
            pltpu.make_async_copy(wa_v, wa_hbm.at[pl.ds(u0, G)], sem),
            pltpu.make_async_copy(wb_v, wb_hbm.at[pl.ds(u0, G)], sem),
        )

    def start(descs):
        for d in descs:
            d.start()

    def wait(descs):
        for d in descs:
            d.wait()

    def gather_group(rlo_v, rhi_v, ilo_v, ihi_v, wa_v, wb_v):
        iota2 = lax.iota(jnp.int32, L) * 2
        for g in range(G):
            gs = jnp.full((L,), g, dtype=jnp.int32)

            @plsc.parallel_loop(0, n_chunks, unroll=8)
            def _(c):
                o = c * L
                idx_lo = ilo_v[g, pl.ds(o, L)]
                idx_hi = ihi_v[g, pl.ds(o, L)]
                a_lo = plsc.load_gather(rlo_v, [gs, idx_lo * 2])
                a_hi = plsc.load_gather(rhi_v, [gs, idx_hi * 2])
                wa_v[g, pl.ds(o, L)] = plsc.bitcast(
                    plsc.pack(a_lo, a_hi, format=plsc.PackFormat.INTERLEAVED),
                    jnp.int32)
                bidx = iota2 + (2 * L * c + 1)
                b_lo = plsc.load_gather(rlo_v, [gs, bidx])
                b_hi = plsc.load_gather(rhi_v, [gs, bidx])
                wb_v[g, pl.ds(o, L)] = plsc.bitcast(
                    plsc.pack(b_lo, b_hi, format=plsc.PackFormat.INTERLEAVED),
                    jnp.int32)

    bufs = ((rlo_v0, rhi_v0, ilo_v0, ihi_v0, wa_v0, wb_v0, sin0, sout0),
            (rlo_v1, rhi_v1, ilo_v1, ihi_v1, wa_v1, wb_v1, sin1, sout1))

    start(in_descs(0, rlo_v0, rhi_v0, ilo_v0, ihi_v0, sin0))

    def pair_body(p, carry):
        for k in (0, 1):
            g = 2 * p + k
            rlo_v, rhi_v, ilo_v, ihi_v, wa_v, wb_v, sin, sout = bufs[k]
            nrlo, nrhi, nilo, nihi, _, _, nsin, _ = bufs[1 - k]

            @pl.when(g + 1 < n_groups)
            def _():
                start(in_descs(g + 1, nrlo, nrhi, nilo, nihi, nsin))

            wait(in_descs(g, rlo_v, rhi_v, ilo_v, ihi_v, sin))

            @pl.when(g >= 2)
            def _():
                wait(out_descs(g - 2, wa_v, wb_v, sout))

            gather_group(rlo_v, rhi_v, ilo_v, ihi_v, wa_v, wb_v)
            start(out_descs(g, wa_v, wb_v, sout))
        return carry

    lax.fori_loop(0, n_groups // 2, pair_body, 0)
    wait(out_descs(n_groups - 2, wa_v0, wb_v0, sout0))
    wait(out_descs(n_groups - 1, wa_v1, wb_v1, sout1))


def _sc_gather(raw2, tr2, row0, nrows):
    _, S2 = raw2.shape
    S = S2 // 2
    half_rows = nrows // 2
    hpw = half_rows // NW
    mesh = plsc.VectorSubcoreMesh(core_axis_name="c", subcore_axis_name="s")
    f = pl.kernel(
        functools.partial(_sc_gather_body, S, half_rows, row0, hpw),
        out_type=[
            jax.ShapeDtypeStruct((half_rows, S), jnp.int32),
            jax.ShapeDtypeStruct((half_rows, S), jnp.int32),
        ],
        mesh=mesh,
        compiler_params=pltpu.CompilerParams(needs_layout_passes=False),
        scratch_types=[
            pltpu.VMEM((G, S2), jnp.float32),
            pltpu.VMEM((G, S2), jnp.float32),
            pltpu.VMEM((G, S), jnp.int32),
            pltpu.VMEM((G, S), jnp.int32),
            pltpu.VMEM((G, S), jnp.int32),
            pltpu.VMEM((G, S), jnp.int32),
            pltpu.VMEM((G, S2), jnp.float32),
            pltpu.VMEM((G, S2), jnp.float32),
            pltpu.VMEM((G, S), jnp.int32),
            pltpu.VMEM((G, S), jnp.int32),
            pltpu.VMEM((G, S), jnp.int32),
            pltpu.VMEM((G, S), jnp.int32),
            pltpu.SemaphoreType.DMA,
            pltpu.SemaphoreType.DMA,
            pltpu.SemaphoreType.DMA,
            pltpu.SemaphoreType.DMA,
        ],
    )
    return f(raw2, tr2)


def _loss_rows(a, b, lab):
    m = jnp.max(jnp.maximum(a, b), axis=1, keepdims=True)
    q = jnp.exp(a - m) + jnp.exp(b - m)
    t = jnp.sum(q, axis=1, keepdims=True)
    logp = jnp.log(q) - jnp.log(t)
    logp = jnp.clip(logp, LOG_EPS, 0.0)
    return -jnp.sum(lab * logp, axis=1, keepdims=True)


def _tc_loss_body(wa_ref, wb_ref, lab_lo_ref, lab_hi_ref, out_lo_ref,
                  out_hi_ref):
    wa = wa_ref[...]
    wb = wb_ref[...]
    a_lo = lax.bitcast_convert_type(wa << 16, jnp.float32)
    a_hi = lax.bitcast_convert_type(wa & jnp.int32(-65536), jnp.float32)
    b_lo = lax.bitcast_convert_type(wb << 16, jnp.float32)
    b_hi = lax.bitcast_convert_type(wb & jnp.int32(-65536), jnp.float32)
    out_lo_ref[...] = _loss_rows(a_lo, b_lo, lab_lo_ref[...])
    out_hi_ref[...] = _loss_rows(a_hi, b_hi, lab_hi_ref[...])


def _tc_loss(wa, wb, labels2, row0):
    half_rows, S = wa.shape
    BR = 128
    grid = (half_rows // BR,)
    off_lo = row0 // BR
    off_hi = (row0 + half_rows) // BR
    out_lo, out_hi = pl.pallas_call(
        _tc_loss_body,
        grid=grid,
        in_specs=[
            pl.BlockSpec((BR, S), lambda i: (i, 0)),
            pl.BlockSpec((BR, S), lambda i: (i, 0)),
            pl.BlockSpec((BR, S), lambda i: (i + off_lo, 0)),
            pl.BlockSpec((BR, S), lambda i: (i + off_hi, 0)),
        ],
        out_specs=[
            pl.BlockSpec((BR, 1), lambda i: (i, 0)),
            pl.BlockSpec((BR, 1), lambda i: (i, 0)),
        ],
        out_shape=[
            jax.ShapeDtypeStruct((half_rows, 1), jnp.float32),
            jax.ShapeDtypeStruct((half_rows, 1), jnp.float32),
        ],
    )(wa, wb, labels2, labels2)
    return out_lo, out_hi


def kernel(raw_logits, target_left, target_right, pointer_labels):
    del target_left
    B, S, S2 = raw_logits.shape
    R = B * S
    raw2 = raw_logits.reshape(R, S2)
    tr2 = target_right.reshape(R, S)
    lab2 = pointer_labels.reshape(R, S)
    cr = R // NCHUNK
    losses = []
    for ci in range(NCHUNK):
        wa, wb = _sc_gather(raw2, tr2, ci * cr, cr)
        lo, hi = _tc_loss(wa, wb, lab2, ci * cr)
        losses.append(lo)
        losses.append(hi)
    loss = jnp.concatenate(losses, axis=0)
    return loss.reshape(B, S)
